# baseline (speedup 1.0000x reference)
import functools

import jax
import jax.numpy as jnp
from jax import lax
from jax.experimental import pallas as pl
from jax.experimental.pallas import tpu as pltpu
from jax.experimental.pallas import tpu_sc as plsc

B, D, K, N, T = 4096, 256, 512, 64, 16
BB = 512
GRID = B // BB
LAMBDA_ORTHO = 0.01

NC, NS = 2, 16
NW = NC * NS
RPW = B // NW
RU = 4


def _bsort(z, sl_ref, lane, asc):
    for dd in (8, 4, 2, 1):
        sl_ref[...] = z
        zp = plsc.load_gather(sl_ref, [jnp.bitwise_xor(lane, dd)])
        lo = (jnp.bitwise_and(lane, dd) == 0)
        mn = jnp.minimum(z, zp)
        mx = jnp.maximum(z, zp)
        z = jnp.where(lo, mn, mx) if asc else jnp.where(lo, mx, mn)
    return z


def _sc_body(g_hbm, out_hbm, g_v, m_v, sh_v, sem):
    wid = lax.axis_index("s") * NC + lax.axis_index("c")
    row0 = wid * RPW
    pltpu.async_copy(g_hbm.at[pl.ds(row0, RPW)], g_v, sem).wait()
    lane = lax.broadcasted_iota(jnp.int32, (16,), 0)

    @plsc.parallel_loop(0, RPW // RU)
    def _rows(i):
        r0 = i * RU
        for j in range(RU):
            r = r0 + j
            vs = []
            for p in range(K // 32):
                ca = g_v[r, pl.ds(32 * p, 16)]
                cb = g_v[r, pl.ds(32 * p + 16, 16)]
                a = plsc.sort_key_val(ca, ca)[0]
                b = plsc.sort_key_val(cb, cb, descending=True)[0]
                vs.append(jnp.minimum(a, b))
            sl = sh_v.at[j]
            while len(vs) > 1:
                nxt = []
                for p in range(len(vs) // 2):
                    x = _bsort(vs[2 * p], sl, lane, True)
                    y = _bsort(vs[2 * p + 1], sl, lane, False)
                    nxt.append(jnp.minimum(x, y))
                vs = nxt
            m_v[r] = vs[0]

    pltpu.sync_copy(m_v, out_hbm.at[pl.ds(row0, RPW)])
    pltpu.sync_copy(m_v, out_hbm.at[pl.ds(row0, RPW)])


@functools.cache
def _make_sc_topk16():
    return pl.kernel(
        _sc_body,
        out_type=jax.ShapeDtypeStruct((B, 16), jnp.float32),
        mesh=plsc.VectorSubcoreMesh(core_axis_name="c", subcore_axis_name="s"),
        scratch_types=[
            pltpu.VMEM((RPW, K), jnp.float32),
            pltpu.VMEM((RPW, 16), jnp.float32),
            pltpu.VMEM((RU, 16), jnp.float32),
            pltpu.SemaphoreType.DMA,
        ],
        compiler_params=pltpu.CompilerParams(needs_layout_passes=False),
    )


def _sc_topk16(g):
    return _make_sc_topk16()(g)


def _tc_a_body(v_ref, vh_ref, f_ref, neg_ref, mask_ref, bv_ref, out_ref):
    i = pl.program_id(0)
    v = v_ref[...]
    vh = vh_ref[...]
    F = f_ref[...]
    neg = neg_ref[...]
    mcol = mask_ref[:, 0:1]

    base = jnp.sqrt(jnp.sum((vh - v) ** 2, axis=1, keepdims=True) + 1e-8)
    vn = jnp.sum(vh * vh, axis=1, keepdims=True)

    nn = jnp.sum(neg * neg, axis=1)
    sneg = jnp.dot(vh, neg.T, preferred_element_type=jnp.float32)
    nd = jnp.sqrt(jnp.maximum(vn - 2.0 * sneg + nn[None, :], 0.0) + 1e-8)
    ju_row = jnp.sum(jnp.maximum(1.0 + base - nd, 0.0), axis=1, keepdims=True) / N

    blanes = lax.broadcasted_iota(jnp.int32, (BB, 128), 1)
    bv_ref[...] = base * (blanes == 0) + vn * (blanes == 1)

    ju_part = jnp.sum(ju_row * mcol)
    mk_part = jnp.sum(mcol)

    lanes = lax.broadcasted_iota(jnp.int32, (1, 1, 128), 2)
    vals = (ju_part * (lanes == 0) + mk_part * (lanes == 2)).astype(jnp.float32)
    out_ref[...] = vals

    @pl.when(i == 0)
    def _ortho():
        gram = jnp.dot(F, F.T, preferred_element_type=jnp.float32)
        r = lax.broadcasted_iota(jnp.int32, (K, K), 0)
        c = lax.broadcasted_iota(jnp.int32, (K, K), 1)
        eye = (r == c).astype(jnp.float32)
        o = jnp.sum(jnp.abs(gram - eye))
        out_ref[...] = vals + o * (lanes == 3)


def _tc_b_body(vh_ref, g_ref, f_ref, mask_ref, th_ref, bv_ref, out_ref):
    vh = vh_ref[...]
    g = g_ref[...]
    F = f_ref[...]
    mcol = mask_ref[:, 0:1]
    base = bv_ref[:, 0:1]
    vn = bv_ref[:, 1:2]

    th = jnp.max(th_ref[...], axis=1, keepdims=True)
    lt = g < th
    cnt = jnp.sum(lt.astype(jnp.float32), axis=1, keepdims=True)
    eq = g == th
    kr = lax.broadcasted_iota(jnp.int32, (K, K), 0)
    kc = lax.broadcasted_iota(jnp.int32, (K, K), 1)
    tri = (kr <= kc).astype(jnp.bfloat16)
    rank = jnp.dot(eq.astype(jnp.bfloat16), tri,
                   preferred_element_type=jnp.float32)
    msel = jnp.logical_or(lt, jnp.logical_and(eq, rank <= (T - cnt)))

    sum_g = jnp.sum(jnp.where(msel, g, 0.0), axis=1, keepdims=True)
    g_t = g / (sum_g + 1e-10)
    m_t = (1.0 - g_t) ** 2

    fn = jnp.sum(F * F, axis=1)
    s = jnp.dot(vh, F.T, preferred_element_type=jnp.float32)
    dft = jnp.sqrt(jnp.maximum(vn - 2.0 * s + fn[None, :], 0.0) + 1e-8)
    hin = jnp.maximum(m_t + base - dft, 0.0)
    jt_row = jnp.sum(jnp.where(msel, hin, 0.0), axis=1, keepdims=True)

    lanes = lax.broadcasted_iota(jnp.int32, (1, 1, 128), 2)
    out_ref[...] = (jnp.sum(jt_row * mcol) * (lanes == 1)).astype(jnp.float32)


def kernel(v, vhat, d, g, F, negatives, mask):
    del d
    cand = _sc_topk16(g)
    mask2 = jnp.broadcast_to(mask.astype(jnp.float32)[:, None], (B, 128))
    bv, parts_a = pl.pallas_call(
        _tc_a_body,
        grid=(GRID,),
        in_specs=[
            pl.BlockSpec((BB, D), lambda i: (i, 0)),
            pl.BlockSpec((BB, D), lambda i: (i, 0)),
            pl.BlockSpec((K, D), lambda i: (0, 0)),
            pl.BlockSpec((N, D), lambda i: (0, 0)),
            pl.BlockSpec((BB, 128), lambda i: (i, 0)),
        ],
        out_specs=[
            pl.BlockSpec((BB, 128), lambda i: (i, 0)),
            pl.BlockSpec((1, 1, 128), lambda i: (i, 0, 0)),
        ],
        out_shape=[
            jax.ShapeDtypeStruct((B, 128), jnp.float32),
            jax.ShapeDtypeStruct((GRID, 1, 128), jnp.float32),
        ],
    )(v, vhat, F, negatives, mask2)
    parts_b = pl.pallas_call(
        _tc_b_body,
        grid=(GRID,),
        in_specs=[
            pl.BlockSpec((BB, D), lambda i: (i, 0)),
            pl.BlockSpec((BB, K), lambda i: (i, 0)),
            pl.BlockSpec((K, D), lambda i: (0, 0)),
            pl.BlockSpec((BB, 128), lambda i: (i, 0)),
            pl.BlockSpec((BB, 16), lambda i: (i, 0)),
            pl.BlockSpec((BB, 128), lambda i: (i, 0)),
        ],
        out_specs=pl.BlockSpec((1, 1, 128), lambda i: (i, 0, 0)),
        out_shape=jax.ShapeDtypeStruct((GRID, 1, 128), jnp.float32),
    )(vhat, g, F, mask2, cand, bv)
    sums = jnp.sum(parts_a + parts_b, axis=(0, 1))
    ju = sums[0] / sums[2]
    jt = sums[1] / jnp.maximum(sums[2], 1.0)
    ortho = sums[3]
    return ju + jt + LAMBDA_ORTHO * ortho ** 2


# speedup vs baseline: 1.4346x; 1.4346x over previous
import functools

import jax
import jax.numpy as jnp
from jax import lax
from jax.experimental import pallas as pl
from jax.experimental.pallas import tpu as pltpu
from jax.experimental.pallas import tpu_sc as plsc

B, D, K, N, T = 4096, 256, 512, 64, 16
BB = 512
GRID = B // BB
LAMBDA_ORTHO = 0.01

NC, NS = 2, 16
NW = NC * NS
RPW = B // NW
RU = 8


def _sc_body(g_hbm, out_hbm, g_v, m_v, sem):
    wid = lax.axis_index("s") * NC + lax.axis_index("c")
    row0 = wid * RPW
    pltpu.async_copy(g_hbm.at[pl.ds(row0, RPW)], g_v, sem).wait()

    @plsc.parallel_loop(0, RPW // RU)
    def _rows(i):
        r0 = i * RU
        runs = []
        for j in range(RU):
            first = g_v[r0 + j, pl.ds(0, 16)]
            runs.append(plsc.sort_key_val(first, first)[0])
        for c in range(1, K // 16):
            for j in range(RU):
                chunk = g_v[r0 + j, pl.ds(16 * c, 16)]
                s_desc = plsc.sort_key_val(chunk, chunk, descending=True)[0]
                merged = jnp.minimum(runs[j], s_desc)
                if c != K // 16 - 1:
                    runs[j] = plsc.sort_key_val(merged, merged)[0]
                else:
                    runs[j] = merged
        for j in range(RU):
            m_v[r0 + j] = runs[j]
    pltpu.sync_copy(m_v, out_hbm.at[pl.ds(row0, RPW)])


@functools.cache
def _make_sc_topk16():
    return pl.kernel(
        _sc_body,
        out_type=jax.ShapeDtypeStruct((B, 16), jnp.float32),
        mesh=plsc.VectorSubcoreMesh(core_axis_name="c", subcore_axis_name="s"),
        scratch_types=[
            pltpu.VMEM((RPW, K), jnp.float32),
            pltpu.VMEM((RPW, 16), jnp.float32),
            pltpu.SemaphoreType.DMA,
        ],
        compiler_params=pltpu.CompilerParams(needs_layout_passes=False),
    )


def _sc_topk16(g):
    return _make_sc_topk16()(g)


def _tc_a_body(v_ref, vh_ref, f_ref, neg_ref, mask_ref, s_ref, bv_ref, out_ref):
    i = pl.program_id(0)
    v = v_ref[...]
    vh = vh_ref[...]
    F = f_ref[...]
    neg = neg_ref[...]
    mcol = mask_ref[:, 0:1]

    base = jnp.sqrt(jnp.sum((vh - v) ** 2, axis=1, keepdims=True) + 1e-8)
    vn = jnp.sum(vh * vh, axis=1, keepdims=True)

    nn = jnp.sum(neg * neg, axis=1)
    sneg = jnp.dot(vh, neg.T, preferred_element_type=jnp.float32)
    nd = jnp.sqrt(jnp.maximum(vn - 2.0 * sneg + nn[None, :], 0.0) + 1e-8)
    ju_row = jnp.sum(jnp.maximum(1.0 + base - nd, 0.0), axis=1, keepdims=True) / N

    s_ref[...] = jnp.dot(vh, F.T, preferred_element_type=jnp.float32)

    blanes = lax.broadcasted_iota(jnp.int32, (BB, 128), 1)
    bv_ref[...] = base * (blanes == 0) + vn * (blanes == 1)

    ju_part = jnp.sum(ju_row * mcol)
    mk_part = jnp.sum(mcol)

    lanes = lax.broadcasted_iota(jnp.int32, (1, 1, 128), 2)
    vals = (ju_part * (lanes == 0) + mk_part * (lanes == 2)).astype(jnp.float32)
    out_ref[...] = vals

    @pl.when(i == 0)
    def _ortho():
        gram = jnp.dot(F, F.T, preferred_element_type=jnp.float32)
        r = lax.broadcasted_iota(jnp.int32, (K, K), 0)
        c = lax.broadcasted_iota(jnp.int32, (K, K), 1)
        eye = (r == c).astype(jnp.float32)
        o = jnp.sum(jnp.abs(gram - eye))
        out_ref[...] = vals + o * (lanes == 3)


def _tc_b_body(s_ref, g_ref, f_ref, mask_ref, th_ref, bv_ref, out_ref):
    g = g_ref[...]
    F = f_ref[...]
    mcol = mask_ref[:, 0:1]
    base = bv_ref[:, 0:1]
    vn = bv_ref[:, 1:2]

    th = jnp.max(th_ref[...], axis=1, keepdims=True)
    lt = g < th
    cnt = jnp.sum(lt.astype(jnp.float32), axis=1, keepdims=True)
    eq = g == th
    kr = lax.broadcasted_iota(jnp.int32, (K, K), 0)
    kc = lax.broadcasted_iota(jnp.int32, (K, K), 1)
    tri = (kr <= kc).astype(jnp.bfloat16)
    rank = jnp.dot(eq.astype(jnp.bfloat16), tri,
                   preferred_element_type=jnp.float32)
    msel = jnp.logical_or(lt, jnp.logical_and(eq, rank <= (T - cnt)))

    sum_g = jnp.sum(jnp.where(msel, g, 0.0), axis=1, keepdims=True)
    g_t = g / (sum_g + 1e-10)
    m_t = (1.0 - g_t) ** 2

    fn = jnp.sum(F * F, axis=1)
    s = s_ref[...]
    dft = jnp.sqrt(jnp.maximum(vn - 2.0 * s + fn[None, :], 0.0) + 1e-8)
    hin = jnp.maximum(m_t + base - dft, 0.0)
    jt_row = jnp.sum(jnp.where(msel, hin, 0.0), axis=1, keepdims=True)

    lanes = lax.broadcasted_iota(jnp.int32, (1, 1, 128), 2)
    out_ref[...] = (jnp.sum(jt_row * mcol) * (lanes == 1)).astype(jnp.float32)


def kernel(v, vhat, d, g, F, negatives, mask):
    del d
    cand = _sc_topk16(g)
    mask2 = jnp.broadcast_to(mask.astype(jnp.float32)[:, None], (B, 128))
    s, bv, parts_a = pl.pallas_call(
        _tc_a_body,
        grid=(GRID,),
        in_specs=[
            pl.BlockSpec((BB, D), lambda i: (i, 0)),
            pl.BlockSpec((BB, D), lambda i: (i, 0)),
            pl.BlockSpec((K, D), lambda i: (0, 0)),
            pl.BlockSpec((N, D), lambda i: (0, 0)),
            pl.BlockSpec((BB, 128), lambda i: (i, 0)),
        ],
        out_specs=[
            pl.BlockSpec((BB, K), lambda i: (i, 0)),
            pl.BlockSpec((BB, 128), lambda i: (i, 0)),
            pl.BlockSpec((1, 1, 128), lambda i: (i, 0, 0)),
        ],
        out_shape=[
            jax.ShapeDtypeStruct((B, K), jnp.float32),
            jax.ShapeDtypeStruct((B, 128), jnp.float32),
            jax.ShapeDtypeStruct((GRID, 1, 128), jnp.float32),
        ],
    )(v, vhat, F, negatives, mask2)
    parts_b = pl.pallas_call(
        _tc_b_body,
        grid=(GRID,),
        in_specs=[
            pl.BlockSpec((BB, K), lambda i: (i, 0)),
            pl.BlockSpec((BB, K), lambda i: (i, 0)),
            pl.BlockSpec((K, D), lambda i: (0, 0)),
            pl.BlockSpec((BB, 128), lambda i: (i, 0)),
            pl.BlockSpec((BB, 16), lambda i: (i, 0)),
            pl.BlockSpec((BB, 128), lambda i: (i, 0)),
        ],
        out_specs=pl.BlockSpec((1, 1, 128), lambda i: (i, 0, 0)),
        out_shape=jax.ShapeDtypeStruct((GRID, 1, 128), jnp.float32),
    )(s, g, F, mask2, cand, bv)
    sums = jnp.sum(parts_a + parts_b, axis=(0, 1))
    ju = sums[0] / sums[2]
    jt = sums[1] / jnp.maximum(sums[2], 1.0)
    ortho = sums[3]
    return ju + jt + LAMBDA_ORTHO * ortho ** 2


# speedup vs baseline: 1.4482x; 1.0095x over previous
import functools

import jax
import jax.numpy as jnp
from jax import lax
from jax.experimental import pallas as pl
from jax.experimental.pallas import tpu as pltpu
from jax.experimental.pallas import tpu_sc as plsc

B, D, K, N, T = 4096, 256, 512, 64, 16
BB = 512
GRID = B // BB
LAMBDA_ORTHO = 0.01

NC, NS = 2, 16
NW = NC * NS
RPW = B // NW
RU = 8


def _sc_body(g_hbm, out_hbm, g_v, m_v, sem):
    wid = lax.axis_index("s") * NC + lax.axis_index("c")
    row0 = wid * RPW
    pltpu.async_copy(g_hbm.at[pl.ds(row0, RPW)], g_v, sem).wait()

    @plsc.parallel_loop(0, RPW // RU)
    def _rows(i):
        r0 = i * RU
        runs = []
        for j in range(RU):
            first = g_v[r0 + j, pl.ds(0, 16)]
            runs.append(plsc.sort_key_val(first, first)[0])
        for c in range(1, K // 16):
            for j in range(RU):
                chunk = g_v[r0 + j, pl.ds(16 * c, 16)]
                s_desc = plsc.sort_key_val(chunk, chunk, descending=True)[0]
                merged = jnp.minimum(runs[j], s_desc)
                if c != K // 16 - 1:
                    runs[j] = plsc.sort_key_val(merged, merged)[0]
                else:
                    runs[j] = merged
        for j in range(RU):
            m_v[r0 + j] = runs[j]
    pltpu.sync_copy(m_v, out_hbm.at[pl.ds(row0, RPW)])


@functools.cache
def _make_sc_topk16():
    return pl.kernel(
        _sc_body,
        out_type=jax.ShapeDtypeStruct((B, 16), jnp.float32),
        mesh=plsc.VectorSubcoreMesh(core_axis_name="c", subcore_axis_name="s"),
        scratch_types=[
            pltpu.VMEM((RPW, K), jnp.float32),
            pltpu.VMEM((RPW, 16), jnp.float32),
            pltpu.SemaphoreType.DMA,
        ],
        compiler_params=pltpu.CompilerParams(needs_layout_passes=False),
    )


def _sc_topk16(g):
    return _make_sc_topk16()(g)


def _tc_a_body(v_ref, vh_ref, f_ref, neg_ref, mask_ref, bv_ref, out_ref):
    i = pl.program_id(0)
    v = v_ref[...]
    vh = vh_ref[...]
    F = f_ref[...]
    neg = neg_ref[...]
    mcol = mask_ref[:, 0:1]

    base = jnp.sqrt(jnp.sum((vh - v) ** 2, axis=1, keepdims=True) + 1e-8)
    vn = jnp.sum(vh * vh, axis=1, keepdims=True)

    nn = jnp.sum(neg * neg, axis=1)
    sneg = jnp.dot(vh, neg.T, preferred_element_type=jnp.float32)
    nd = jnp.sqrt(jnp.maximum(vn - 2.0 * sneg + nn[None, :], 0.0) + 1e-8)
    ju_row = jnp.sum(jnp.maximum(1.0 + base - nd, 0.0), axis=1, keepdims=True) / N

    blanes = lax.broadcasted_iota(jnp.int32, (BB, 128), 1)
    bv_ref[...] = base * (blanes == 0) + vn * (blanes == 1)

    ju_part = jnp.sum(ju_row * mcol)
    mk_part = jnp.sum(mcol)

    lanes = lax.broadcasted_iota(jnp.int32, (1, 1, 128), 2)
    vals = (ju_part * (lanes == 0) + mk_part * (lanes == 2)).astype(jnp.float32)
    out_ref[...] = vals

    @pl.when(i == 0)
    def _ortho():
        gram = jnp.dot(F, F.T, preferred_element_type=jnp.float32)
        r = lax.broadcasted_iota(jnp.int32, (K, K), 0)
        c = lax.broadcasted_iota(jnp.int32, (K, K), 1)
        eye = (r == c).astype(jnp.float32)
        o = jnp.sum(jnp.abs(gram - eye))
        out_ref[...] = vals + o * (lanes == 3)


def _tc_b_body(vh_ref, g_ref, f_ref, mask_ref, th_ref, bv_ref, out_ref):
    vh = vh_ref[...]
    g = g_ref[...]
    F = f_ref[...]
    mcol = mask_ref[:, 0:1]
    base = bv_ref[:, 0:1]
    vn = bv_ref[:, 1:2]

    th = jnp.max(th_ref[...], axis=1, keepdims=True)
    lt = g < th
    cnt = jnp.sum(lt.astype(jnp.float32), axis=1, keepdims=True)
    eq = g == th
    kr = lax.broadcasted_iota(jnp.int32, (K, K), 0)
    kc = lax.broadcasted_iota(jnp.int32, (K, K), 1)
    tri = (kr <= kc).astype(jnp.bfloat16)
    rank = jnp.dot(eq.astype(jnp.bfloat16), tri,
                   preferred_element_type=jnp.float32)
    msel = jnp.logical_or(lt, jnp.logical_and(eq, rank <= (T - cnt)))

    sum_g = jnp.sum(jnp.where(msel, g, 0.0), axis=1, keepdims=True)
    g_t = g / (sum_g + 1e-10)
    m_t = (1.0 - g_t) ** 2

    fn = jnp.sum(F * F, axis=1)
    s = jnp.dot(vh, F.T, preferred_element_type=jnp.float32)
    dft = jnp.sqrt(jnp.maximum(vn - 2.0 * s + fn[None, :], 0.0) + 1e-8)
    hin = jnp.maximum(m_t + base - dft, 0.0)
    jt_row = jnp.sum(jnp.where(msel, hin, 0.0), axis=1, keepdims=True)

    lanes = lax.broadcasted_iota(jnp.int32, (1, 1, 128), 2)
    out_ref[...] = (jnp.sum(jt_row * mcol) * (lanes == 1)).astype(jnp.float32)


def kernel(v, vhat, d, g, F, negatives, mask):
    del d
    cand = _sc_topk16(g)
    mask2 = jnp.broadcast_to(mask.astype(jnp.float32)[:, None], (B, 128))
    bv, parts_a = pl.pallas_call(
        _tc_a_body,
        grid=(GRID,),
        in_specs=[
            pl.BlockSpec((BB, D), lambda i: (i, 0)),
            pl.BlockSpec((BB, D), lambda i: (i, 0)),
            pl.BlockSpec((K, D), lambda i: (0, 0)),
            pl.BlockSpec((N, D), lambda i: (0, 0)),
            pl.BlockSpec((BB, 128), lambda i: (i, 0)),
        ],
        out_specs=[
            pl.BlockSpec((BB, 128), lambda i: (i, 0)),
            pl.BlockSpec((1, 1, 128), lambda i: (i, 0, 0)),
        ],
        out_shape=[
            jax.ShapeDtypeStruct((B, 128), jnp.float32),
            jax.ShapeDtypeStruct((GRID, 1, 128), jnp.float32),
        ],
    )(v, vhat, F, negatives, mask2)
    parts_b = pl.pallas_call(
        _tc_b_body,
        grid=(GRID,),
        in_specs=[
            pl.BlockSpec((BB, D), lambda i: (i, 0)),
            pl.BlockSpec((BB, K), lambda i: (i, 0)),
            pl.BlockSpec((K, D), lambda i: (0, 0)),
            pl.BlockSpec((BB, 128), lambda i: (i, 0)),
            pl.BlockSpec((BB, 16), lambda i: (i, 0)),
            pl.BlockSpec((BB, 128), lambda i: (i, 0)),
        ],
        out_specs=pl.BlockSpec((1, 1, 128), lambda i: (i, 0, 0)),
        out_shape=jax.ShapeDtypeStruct((GRID, 1, 128), jnp.float32),
    )(vhat, g, F, mask2, cand, bv)
    sums = jnp.sum(parts_a + parts_b, axis=(0, 1))
    ju = sums[0] / sums[2]
    jt = sums[1] / jnp.maximum(sums[2], 1.0)
    ortho = sums[3]
    return ju + jt + LAMBDA_ORTHO * ortho ** 2
